# baseline (speedup 1.0000x reference)
import functools

import jax
import jax.numpy as jnp
from jax import lax
from jax.experimental import pallas as pl
from jax.experimental.pallas import tpu as pltpu
from jax.experimental.pallas import tpu_sc as plsc

NUM_RELATIONS = 1000000
EMBEDDING_DIM = 32
BATCH = 4096
FIELDS = 26

_info = plsc.get_sparse_core_info()
_NC, _NS = _info.num_cores, _info.num_subcores
_NW = _NC * _NS
_NROWS = FIELDS * EMBEDDING_DIM
_RPW = _NROWS // _NW
_NS_W = 8
_CS = BATCH // _NS_W
_NG = _CS // 16
_BPR = NUM_RELATIONS // 16


@functools.partial(
    pl.kernel,
    out_type=jax.ShapeDtypeStruct((FIELDS, EMBEDDING_DIM, BATCH), jnp.float32),
    mesh=plsc.VectorSubcoreMesh(core_axis_name="c", subcore_axis_name="s"),
    scratch_types=[
        pltpu.VMEM((_NS_W, _CS), jnp.int32),
        pltpu.VMEM((_NS_W, _CS), jnp.int32),
        pltpu.VMEM((_NS_W, _CS, 16), jnp.float32),
        pltpu.VMEM((_NS_W, _CS), jnp.float32),
        pltpu.SemaphoreType.DMA((_NS_W,)),
        pltpu.SemaphoreType.DMA((_NS_W,)),
        pltpu.SemaphoreType.DMA((_NS_W,)),
    ],
    compiler_params=pltpu.CompilerParams(
        use_tc_tiling_on_sc=False, needs_layout_passes=False
    ),
)
def _gather_kernel(
    tab_hbm, ids_hbm, out_hbm, idx_v, bidx_v, stg_v, row_v, isems, gsems, wsems
):
    wid = lax.axis_index("s") * _NC + lax.axis_index("c")
    r0 = wid * _RPW
    lane = lax.iota(jnp.int32, 16)

    def wave(w, _):
        r = r0 + w
        f = r // EMBEDDING_DIM
        d = r % EMBEDDING_DIM
        doff = d * _BPR

        ihs = []
        for q in range(_NS_W):
            ihs.append(
                pltpu.async_copy(
                    ids_hbm.at[f, pl.ds(q * _CS, _CS)], idx_v.at[q], isems.at[q]
                )
            )
        ghs = []
        for q in range(_NS_W):
            ihs[q].wait()
            for g in range(_NG):
                sl = pl.ds(g * 16, 16)
                bidx_v[q, sl] = lax.shift_right_logical(idx_v[q, sl], 4) + doff
            ghs.append(
                pltpu.async_copy(
                    tab_hbm.at[bidx_v.at[q]], stg_v.at[q], gsems.at[q]
                )
            )
        whs = []
        for q in range(_NS_W):
            ghs[q].wait()

            def sel(g, _, q=q):
                sl = pl.ds(g * 16, 16)
                low = lax.bitwise_and(idx_v[q, sl], 15)
                row_v[q, sl] = plsc.load_gather(stg_v.at[q], [g * 16 + lane, low])
                return ()

            lax.fori_loop(0, _NG, sel, ())
            whs.append(
                pltpu.async_copy(
                    row_v.at[q], out_hbm.at[f, d, pl.ds(q * _CS, _CS)], wsems.at[q]
                )
            )
        for q in range(_NS_W):
            whs[q].wait()
        return ()

    lax.fori_loop(0, _RPW, wave, ())


@jax.jit
def kernel(relation_ids, embedding_table):
    tab4 = embedding_table.T.reshape(EMBEDDING_DIM * NUM_RELATIONS // 16, 16)
    ids_t = relation_ids.T.astype(jnp.int32)
    out = _gather_kernel(tab4, ids_t)
    return out.transpose(2, 0, 1)
